# baseline (speedup 1.0000x reference)
import functools

import jax
import jax.numpy as jnp
from jax import lax
from jax.experimental import pallas as pl
from jax.experimental.pallas import tpu as pltpu
from jax.experimental.pallas import tpu_sc as plsc

_EPS_LN = 1e-5
_CHUNK = 128


def _silu(z):
    return z * jax.nn.sigmoid(z)


def _iln(h, g, b):
    m = jnp.mean(h, axis=-1, keepdims=True)
    v = jnp.mean((h - m) * (h - m), axis=-1, keepdims=True)
    return (h - m) / jnp.sqrt(v + _EPS_LN) * g + b


def _dot(a, b):
    return jnp.dot(a, b, preferred_element_type=jnp.float32)


def _sc_segsum(values, j_flat, n_nodes):
    e_total, width = values.shape
    n_chunks = e_total // _CHUNK
    info = plsc.get_sparse_core_info()
    nc, ns = info.num_cores, info.num_subcores
    nw = nc * ns
    trips = (n_chunks + nw - 1) // nw
    rows_per_tile = (((n_nodes + ns - 1) // ns) + 7) // 8 * 8
    n_pad = rows_per_tile * ns
    zeros = jnp.zeros((n_pad, 128), jnp.float32)
    mesh = plsc.VectorSubcoreMesh(core_axis_name="c", subcore_axis_name="s")

    n_slices = width // 16

    @functools.partial(
        pl.kernel,
        out_type=jax.ShapeDtypeStruct((nc, n_pad, 128), jnp.float32),
        mesh=mesh,
        scratch_types=[
            pltpu.VMEM((_CHUNK,), jnp.int32),
            pltpu.VMEM((_CHUNK,), jnp.int32),
            pltpu.VMEM((_CHUNK, width), jnp.float32),
            pltpu.VMEM((_CHUNK, width), jnp.float32),
            pltpu.VMEM((_CHUNK, 128), jnp.float32),
            pltpu.VMEM((_CHUNK,), jnp.int32),
            pltpu.VMEM_SHARED((n_pad, 128), jnp.float32),
            pltpu.SemaphoreType.DMA,
            pltpu.SemaphoreType.DMA,
            pltpu.SemaphoreType.DMA,
            pltpu.SemaphoreType.DMA,
        ],
    )
    def body(vals_hbm, j_hbm, zeros_hbm, out_hbm, idx0, idx1, val0, val1,
             wide_s, idx_s, acc, si0, si1, sv0, sv1):
        idx = (idx0, idx1)
        val = (val0, val1)
        semi = (si0, si1)
        semv = (sv0, sv1)
        cid = lax.axis_index("c")
        sid = lax.axis_index("s")
        w = sid * nc + cid
        r0 = sid * rows_per_tile
        pltpu.sync_copy(zeros_hbm.at[pl.ds(r0, rows_per_tile)],
                        acc.at[pl.ds(r0, rows_per_tile)])
        pltpu.sync_copy(zeros_hbm.at[pl.ds(0, _CHUNK)], wide_s)
        plsc.subcore_barrier()

        def fire(t, b):
            c = w + t * nw

            @pl.when(c < n_chunks)
            def _():
                pltpu.async_copy(j_hbm.at[pl.ds(c * _CHUNK, _CHUNK)],
                                 idx[b], semi[b])
                pltpu.async_copy(vals_hbm.at[pl.ds(c * _CHUNK, _CHUNK)],
                                 val[b], semv[b])

        def proc(t, b):
            c = w + t * nw

            @pl.when(c < n_chunks)
            def _():
                pltpu.make_async_copy(
                    j_hbm.at[pl.ds(c * _CHUNK, _CHUNK)], idx[b],
                    semi[b]).wait()
                pltpu.make_async_copy(
                    vals_hbm.at[pl.ds(c * _CHUNK, _CHUNK)], val[b],
                    semv[b]).wait()

                def widen_row(r, cc):
                    for k in range(n_slices):
                        sl = pl.ds(k * 16, 16)
                        wide_s[r, sl] = val[b][r, sl]
                    return cc

                lax.fori_loop(0, _CHUNK, widen_row, 0)

                def copy_idx(_):
                    for k in range(_CHUNK // 16):
                        sl = pl.ds(k * 16, 16)
                        idx_s[sl] = idx[b][sl]
                copy_idx(None)
                pltpu.sync_copy(wide_s, acc.at[idx_s], add=True)

        fire(0, 0)

        def step(tt, carry):
            t = tt * 2
            fire(t + 1, 1)
            proc(t, 0)
            fire(t + 2, 0)
            proc(t + 1, 1)
            return carry

        lax.fori_loop(0, (trips + 1) // 2, step, 0)
        plsc.subcore_barrier()
        pltpu.sync_copy(acc.at[pl.ds(r0, rows_per_tile)],
                        out_hbm.at[cid].at[pl.ds(r0, rows_per_tile)])

    return body(values, j_flat, zeros)


def _sc_gather_add(a_tab, b_tab, i_flat, j_flat, used_width):
    e_total = i_flat.shape[0]
    width = a_tab.shape[1]
    n_chunks = e_total // _CHUNK
    info = plsc.get_sparse_core_info()
    nc, ns = info.num_cores, info.num_subcores
    nw = nc * ns
    trips = (n_chunks + nw - 1) // nw
    n_slices = used_width // 16
    mesh = plsc.VectorSubcoreMesh(core_axis_name="c", subcore_axis_name="s")

    @functools.partial(
        pl.kernel,
        out_type=jax.ShapeDtypeStruct((e_total, width), jnp.float32),
        mesh=mesh,
        scratch_types=[
            pltpu.VMEM((_CHUNK,), jnp.int32),
            pltpu.VMEM((_CHUNK,), jnp.int32),
            pltpu.VMEM((_CHUNK,), jnp.int32),
            pltpu.VMEM((_CHUNK,), jnp.int32),
            pltpu.VMEM((_CHUNK, width), jnp.float32),
            pltpu.VMEM((_CHUNK, width), jnp.float32),
            pltpu.VMEM((_CHUNK, width), jnp.float32),
            pltpu.VMEM((_CHUNK, width), jnp.float32),
            pltpu.SemaphoreType.DMA,
            pltpu.SemaphoreType.DMA,
            pltpu.SemaphoreType.DMA,
            pltpu.SemaphoreType.DMA,
        ],
    )
    def body(a_hbm, b_hbm, i_hbm, j_hbm, out_hbm,
             ii0, ii1, jj0, jj1, ba0, ba1, bb0, bb1, sa0, sa1, sb0, sb1):
        ii = (ii0, ii1)
        jj = (jj0, jj1)
        buf_a = (ba0, ba1)
        buf_b = (bb0, bb1)
        sem_a = (sa0, sa1)
        sem_b = (sb0, sb1)
        cid = lax.axis_index("c")
        sid = lax.axis_index("s")
        w = sid * nc + cid

        def fire(t, b):
            c = w + t * nw

            @pl.when(c < n_chunks)
            def _():
                pltpu.sync_copy(i_hbm.at[pl.ds(c * _CHUNK, _CHUNK)], ii[b])
                pltpu.sync_copy(j_hbm.at[pl.ds(c * _CHUNK, _CHUNK)], jj[b])
                pltpu.async_copy(a_hbm.at[ii[b]], buf_a[b], sem_a[b])
                pltpu.async_copy(b_hbm.at[jj[b]], buf_b[b], sem_b[b])

        def proc(t, b):
            c = w + t * nw

            @pl.when(c < n_chunks)
            def _():
                pltpu.make_async_copy(a_hbm.at[ii[b]], buf_a[b],
                                      sem_a[b]).wait()
                pltpu.make_async_copy(b_hbm.at[jj[b]], buf_b[b],
                                      sem_b[b]).wait()

                def add_row(r, cc):
                    for k in range(n_slices):
                        sl = pl.ds(k * 16, 16)
                        buf_a[b][r, sl] = buf_a[b][r, sl] + buf_b[b][r, sl]
                    return cc

                lax.fori_loop(0, _CHUNK, add_row, 0)
                pltpu.sync_copy(buf_a[b],
                                out_hbm.at[pl.ds(c * _CHUNK, _CHUNK)])

        fire(0, 0)

        def step(tt, carry):
            t = tt * 2
            fire(t + 1, 1)
            proc(t, 0)
            fire(t + 2, 0)
            proc(t + 1, 1)
            return carry

        lax.fori_loop(0, (trips + 1) // 2, step, 0)

    return body(a_tab, b_tab, i_flat, j_flat)


def _tc_node(node_h, x, part, wts):
    n, node_dim = node_h.shape
    blk = 2000
    grid = n // blk

    def body(nh_ref, x_ref, part_ref, w1_ref, b1_ref, w2_ref, b2_ref,
             w3_ref, b3_ref, g_ref, beta_ref, wa_ref, wb_ref,
             out_h2, out_a, out_b):
        agg = (part_ref[0] + part_ref[1])[:, :16]
        h0 = nh_ref[...]
        w1 = w1_ref[...]
        z = _dot(h0, w1[:node_dim]) + _dot(agg, w1[node_dim:]) + b1_ref[...]
        z = _silu(z)
        z = _silu(_dot(z, w2_ref[...]) + b2_ref[...])
        h = _dot(z, w3_ref[...]) + b3_ref[...]
        h2 = _iln(h, g_ref[...], beta_ref[...]) + h0
        out_h2[...] = h2
        xb = x_ref[...]
        xpad = jnp.concatenate(
            [xb, jnp.zeros((blk, 61), jnp.float32)], axis=-1)
        out_a[...] = jnp.concatenate([_dot(h2, wa_ref[...]), -xpad], axis=-1)
        out_b[...] = jnp.concatenate([_dot(h2, wb_ref[...]), xpad], axis=-1)

    full = lambda s: pl.BlockSpec(s, lambda i: tuple(0 for _ in s))
    row = lambda s: pl.BlockSpec(s, lambda i: (i,) + tuple(0 for _ in s[1:]))
    w1, b1, w2, b2, w3, b3, g, beta, wa, wb = wts
    return pl.pallas_call(
        body,
        grid=(grid,),
        in_specs=[
            row((blk, node_dim)),
            row((blk, 3)),
            pl.BlockSpec((2, blk, 128), lambda i: (0, i, 0)),
            full(w1.shape), full(b1.shape), full(w2.shape), full(b2.shape),
            full(w3.shape), full(b3.shape), full(g.shape), full(beta.shape),
            full(wa.shape), full(wb.shape),
        ],
        out_specs=[
            row((blk, node_dim)), row((blk, 128)), row((blk, 128)),
        ],
        out_shape=[
            jax.ShapeDtypeStruct((n, node_dim), jnp.float32),
            jax.ShapeDtypeStruct((n, 128), jnp.float32),
            jax.ShapeDtypeStruct((n, 128), jnp.float32),
        ],
    )(node_h, x, part, w1, b1, w2, b2, w3, b3, g, beta, wa, wb)


def _tc_edge(sd, et, wts):
    e_total = sd.shape[0]
    blk = 3200
    grid = e_total // blk
    (w1cw, bias1, w2blk, bias2, w3blk, bias3,
     w1big, g128, beta128, eo_w, eo_bias) = wts

    def body(sd_ref, et_ref, w1cw_r, bias1_r, w2_r, bias2_r, w3_r, bias3_r,
             w1big_r, g128_r, beta128_r, eow_r, eobias_r, out_ref):
        sdb = sd_ref[...]
        d16 = sdb[:, 64:80]
        dl = jnp.sqrt(jnp.sum(d16 * d16, axis=-1, keepdims=True) + 1e-12)
        lane = lax.broadcasted_iota(jnp.int32, sdb.shape, 1)
        sd_d = jnp.where(lane == 67, dl, sdb)
        s_only = jnp.where(lane < 64, sdb, 0.0)
        z = (s_only + _dot(et_ref[...], w1cw_r[...])
             + _dot(sd_d, w1big_r[...]) + bias1_r[...])
        uv = _silu(z)
        uv = _silu(_dot(uv, w2_r[...]) + bias2_r[...])
        pre = _dot(uv, w3_r[...]) + bias3_r[...]
        m1 = jnp.mean(pre[:, :64], axis=-1, keepdims=True)
        m2 = jnp.mean(pre[:, 64:], axis=-1, keepdims=True)
        m = jnp.where(lane < 64, m1, m2)
        cen = pre - m
        v1 = jnp.mean(cen[:, :64] * cen[:, :64], axis=-1, keepdims=True)
        v2 = jnp.mean(cen[:, 64:] * cen[:, 64:], axis=-1, keepdims=True)
        vv = jnp.where(lane < 64, v1, v2)
        pre_n = cen / jnp.sqrt(vv + _EPS_LN) * g128_r[...] + beta128_r[...]
        out_ref[...] = _dot(pre_n, eow_r[...]) + eobias_r[...]

    full = lambda s: pl.BlockSpec(s, lambda i: tuple(0 for _ in s))
    row = lambda s: pl.BlockSpec(s, lambda i: (i,) + tuple(0 for _ in s[1:]))
    return pl.pallas_call(
        body,
        grid=(grid,),
        in_specs=[row((blk, 128)), row((blk, 16))] + [full(w.shape) for w in (
            w1cw, bias1, w2blk, bias2, w3blk, bias3,
            w1big, g128, beta128, eo_w, eo_bias)],
        out_specs=[row((blk, 64))],
        out_shape=[jax.ShapeDtypeStruct((e_total, 64), jnp.float32)],
    )(sd, et, w1cw, bias1, w2blk, bias2, w3blk, bias3,
      w1big, g128, beta128, eo_w, eo_bias)[0]


def _tc_pos(x, part, wts):
    n = x.shape[0]
    blk = 2000
    grid = n // blk
    w1x, w1a, b1, w2, b2, w3, b3, g, beta = wts

    def body(x_ref, part_ref, w1x_r, w1a_r, b1_r, w2_r, b2_r, w3_r, b3_r,
             g_r, beta_r, out_ref):
        agg = (part_ref[0] + part_ref[1])[:, :64]
        xb = x_ref[...]
        z = _dot(xb, w1x_r[...]) + _dot(agg, w1a_r[...]) + b1_r[...]
        z = _silu(z)
        z = _silu(_dot(z, w2_r[...]) + b2_r[...])
        t = _dot(z, w3_r[...]) + b3_r[...]
        out_ref[...] = _iln(t, g_r[...], beta_r[...]) + xb

    full = lambda s: pl.BlockSpec(s, lambda i: tuple(0 for _ in s))
    row = lambda s: pl.BlockSpec(s, lambda i: (i,) + tuple(0 for _ in s[1:]))
    return pl.pallas_call(
        body,
        grid=(grid,),
        in_specs=[
            row((blk, 3)),
            pl.BlockSpec((2, blk, 128), lambda i: (0, i, 0)),
            full(w1x.shape), full(w1a.shape), full(b1.shape), full(w2.shape),
            full(b2.shape), full(w3.shape), full(b3.shape), full(g.shape),
            full(beta.shape),
        ],
        out_specs=[row((blk, 3))],
        out_shape=[jax.ShapeDtypeStruct((n, 3), jnp.float32)],
    )(x, part, w1x, w1a, b1, w2, b2, w3, b3, g, beta)[0]


def kernel(node_h, x, edge_index, edge_type_h, params):
    n, node_dim = node_h.shape
    e_total = edge_index.shape[1]
    i_flat = edge_index[0]
    j_flat = edge_index[1]

    r2 = lambda v: v.reshape(1, -1)

    part1 = _sc_segsum(edge_type_h, j_flat, n)

    ehw1 = params['eh_W'][0]
    node_wts = (
        params['n_W'][0], r2(params['n_b'][0]),
        params['n_W'][1], r2(params['n_b'][1]),
        params['n_W'][2], r2(params['n_b'][2]),
        r2(params['n_g']), r2(params['n_beta']),
        ehw1[:node_dim], ehw1[node_dim:2 * node_dim],
    )
    node_h2, a_tab, b_tab = _tc_node(node_h, x, part1, node_wts)

    sd = _sc_gather_add(a_tab, b_tab, i_flat, j_flat, 80)

    exw1 = params['ex_W'][0]
    xw1 = jnp.zeros((16, 64), jnp.float32).at[:4].set(exw1)
    def bdiag(a, b):
        za = jnp.zeros((a.shape[0], b.shape[1]), jnp.float32)
        zb = jnp.zeros((b.shape[0], a.shape[1]), jnp.float32)
        return jnp.concatenate([jnp.concatenate([a, za], -1),
                                jnp.concatenate([zb, b], -1)], 0)

    w1cw = jnp.concatenate(
        [ehw1[2 * node_dim:], jnp.zeros((16, 64), jnp.float32)], axis=1)
    w1big = jnp.zeros((128, 128), jnp.float32).at[64:80, 64:].set(xw1)
    w2blk = bdiag(params['eh_W'][1], params['ex_W'][1])
    w3blk = bdiag(params['eh_W'][2], params['ex_W'][2])
    cat2 = lambda a, b: jnp.concatenate([a, b]).reshape(1, -1)
    edge_wts = (
        w1cw, cat2(params['eh_b'][0], params['ex_b'][0]),
        w2blk, cat2(params['eh_b'][1], params['ex_b'][1]),
        w3blk, cat2(params['eh_b'][2], params['ex_b'][2]),
        w1big,
        cat2(params['eh_g'], params['ex_g']),
        cat2(params['eh_beta'], params['ex_beta']),
        params['eo_W'], r2(params['eo_b']),
    )
    edge_h = _tc_edge(sd, edge_type_h, edge_wts)

    part2 = _sc_segsum(edge_h, j_flat, n)

    pw1 = params['p_W'][0]
    pos_wts = (
        pw1[:3], pw1[3:], r2(params['p_b'][0]),
        params['p_W'][1], r2(params['p_b'][1]),
        params['p_W'][2], r2(params['p_b'][2]),
        r2(params['p_g']), r2(params['p_beta']),
    )
    x2 = _tc_pos(x, part2, pos_wts)

    return (edge_h, node_h2, x2)


# speedup vs baseline: 1.1516x; 1.1516x over previous
import functools

import jax
import jax.numpy as jnp
from jax import lax
from jax.experimental import pallas as pl
from jax.experimental.pallas import tpu as pltpu
from jax.experimental.pallas import tpu_sc as plsc

_EPS_LN = 1e-5
_CHUNK = 128


def _silu(z):
    return z * jax.nn.sigmoid(z)


def _iln(h, g, b):
    m = jnp.mean(h, axis=-1, keepdims=True)
    v = jnp.mean((h - m) * (h - m), axis=-1, keepdims=True)
    return (h - m) / jnp.sqrt(v + _EPS_LN) * g + b


def _dot(a, b):
    return jnp.dot(a, b, preferred_element_type=jnp.float32)


def _sc_segsum(values, j_flat, n_nodes):
    e_total, width = values.shape
    n_chunks = e_total // _CHUNK
    info = plsc.get_sparse_core_info()
    nc, ns = info.num_cores, info.num_subcores
    nw = nc * ns
    trips = (n_chunks + nw - 1) // nw
    rows_per_tile = (((n_nodes + ns - 1) // ns) + 7) // 8 * 8
    n_pad = rows_per_tile * ns
    zeros = jnp.zeros((n_pad, 128), jnp.float32)
    mesh = plsc.VectorSubcoreMesh(core_axis_name="c", subcore_axis_name="s")

    n_slices = width // 16

    @functools.partial(
        pl.kernel,
        out_type=jax.ShapeDtypeStruct((nc, n_pad, 128), jnp.float32),
        mesh=mesh,
        scratch_types=[
            pltpu.VMEM((_CHUNK,), jnp.int32),
            pltpu.VMEM((_CHUNK,), jnp.int32),
            pltpu.VMEM((_CHUNK, width), jnp.float32),
            pltpu.VMEM((_CHUNK, width), jnp.float32),
            pltpu.VMEM((_CHUNK, 128), jnp.float32),
            pltpu.VMEM((_CHUNK,), jnp.int32),
            pltpu.VMEM_SHARED((n_pad, 128), jnp.float32),
            pltpu.SemaphoreType.DMA,
            pltpu.SemaphoreType.DMA,
            pltpu.SemaphoreType.DMA,
            pltpu.SemaphoreType.DMA,
        ],
    )
    def body(vals_hbm, j_hbm, zeros_hbm, out_hbm, idx0, idx1, val0, val1,
             wide_s, idx_s, acc, si0, si1, sv0, sv1):
        idx = (idx0, idx1)
        val = (val0, val1)
        semi = (si0, si1)
        semv = (sv0, sv1)
        cid = lax.axis_index("c")
        sid = lax.axis_index("s")
        w = sid * nc + cid
        r0 = sid * rows_per_tile
        pltpu.sync_copy(zeros_hbm.at[pl.ds(r0, rows_per_tile)],
                        acc.at[pl.ds(r0, rows_per_tile)])
        pltpu.sync_copy(zeros_hbm.at[pl.ds(0, _CHUNK)], wide_s)
        plsc.subcore_barrier()

        def fire(t, b):
            c = w + t * nw

            @pl.when(c < n_chunks)
            def _():
                pltpu.async_copy(j_hbm.at[pl.ds(c * _CHUNK, _CHUNK)],
                                 idx[b], semi[b])
                pltpu.async_copy(vals_hbm.at[pl.ds(c * _CHUNK, _CHUNK)],
                                 val[b], semv[b])

        def proc(t, b):
            c = w + t * nw

            @pl.when(c < n_chunks)
            def _():
                pltpu.make_async_copy(
                    j_hbm.at[pl.ds(c * _CHUNK, _CHUNK)], idx[b],
                    semi[b]).wait()
                pltpu.make_async_copy(
                    vals_hbm.at[pl.ds(c * _CHUNK, _CHUNK)], val[b],
                    semv[b]).wait()

                def widen_row(r, cc):
                    for k in range(n_slices):
                        sl = pl.ds(k * 16, 16)
                        wide_s[r, sl] = val[b][r, sl]
                    return cc

                lax.fori_loop(0, _CHUNK, widen_row, 0)

                def copy_idx(_):
                    for k in range(_CHUNK // 16):
                        sl = pl.ds(k * 16, 16)
                        idx_s[sl] = idx[b][sl]
                copy_idx(None)
                pltpu.sync_copy(wide_s, acc.at[idx_s], add=True)

        fire(0, 0)

        def step(tt, carry):
            t = tt * 2
            fire(t + 1, 1)
            proc(t, 0)
            fire(t + 2, 0)
            proc(t + 1, 1)
            return carry

        lax.fori_loop(0, (trips + 1) // 2, step, 0)
        plsc.subcore_barrier()
        pltpu.sync_copy(acc.at[pl.ds(r0, rows_per_tile)],
                        out_hbm.at[cid].at[pl.ds(r0, rows_per_tile)])

    return body(values, j_flat, zeros)


def _sc_gather_add(a_tab, b_tab, i_flat, j_flat, used_width):
    e_total = i_flat.shape[0]
    width = a_tab.shape[1]
    n_chunks = e_total // _CHUNK
    info = plsc.get_sparse_core_info()
    nc, ns = info.num_cores, info.num_subcores
    nw = nc * ns
    trips = (n_chunks + nw - 1) // nw
    n_slices = used_width // 16
    mesh = plsc.VectorSubcoreMesh(core_axis_name="c", subcore_axis_name="s")

    @functools.partial(
        pl.kernel,
        out_type=jax.ShapeDtypeStruct((e_total, width), jnp.float32),
        mesh=mesh,
        scratch_types=[
            pltpu.VMEM((_CHUNK,), jnp.int32),
            pltpu.VMEM((_CHUNK,), jnp.int32),
            pltpu.VMEM((_CHUNK,), jnp.int32),
            pltpu.VMEM((_CHUNK,), jnp.int32),
            pltpu.VMEM((_CHUNK, width), jnp.float32),
            pltpu.VMEM((_CHUNK, width), jnp.float32),
            pltpu.VMEM((_CHUNK, width), jnp.float32),
            pltpu.VMEM((_CHUNK, width), jnp.float32),
            pltpu.SemaphoreType.DMA,
            pltpu.SemaphoreType.DMA,
            pltpu.SemaphoreType.DMA,
            pltpu.SemaphoreType.DMA,
        ],
    )
    def body(a_hbm, b_hbm, i_hbm, j_hbm, out_hbm,
             ii0, ii1, jj0, jj1, ba0, ba1, bb0, bb1, sa0, sa1, sb0, sb1):
        ii = (ii0, ii1)
        jj = (jj0, jj1)
        buf_a = (ba0, ba1)
        buf_b = (bb0, bb1)
        sem_a = (sa0, sa1)
        sem_b = (sb0, sb1)
        cid = lax.axis_index("c")
        sid = lax.axis_index("s")
        w = sid * nc + cid

        def fire(t, b):
            c = w + t * nw

            @pl.when(c < n_chunks)
            def _():
                pltpu.sync_copy(i_hbm.at[pl.ds(c * _CHUNK, _CHUNK)], ii[b])
                pltpu.sync_copy(j_hbm.at[pl.ds(c * _CHUNK, _CHUNK)], jj[b])
                pltpu.async_copy(a_hbm.at[ii[b]], buf_a[b], sem_a[b])
                pltpu.async_copy(b_hbm.at[jj[b]], buf_b[b], sem_b[b])

        def proc(t, b):
            c = w + t * nw

            @pl.when(c < n_chunks)
            def _():
                pltpu.make_async_copy(a_hbm.at[ii[b]], buf_a[b],
                                      sem_a[b]).wait()
                pltpu.make_async_copy(b_hbm.at[jj[b]], buf_b[b],
                                      sem_b[b]).wait()

                def add_row(r, cc):
                    for k in range(n_slices):
                        sl = pl.ds(k * 16, 16)
                        buf_a[b][r, sl] = buf_a[b][r, sl] + buf_b[b][r, sl]
                    return cc

                lax.fori_loop(0, _CHUNK, add_row, 0)
                pltpu.sync_copy(buf_a[b],
                                out_hbm.at[pl.ds(c * _CHUNK, _CHUNK)])

        fire(0, 0)

        def step(tt, carry):
            t = tt * 2
            fire(t + 1, 1)
            proc(t, 0)
            fire(t + 2, 0)
            proc(t + 1, 1)
            return carry

        lax.fori_loop(0, (trips + 1) // 2, step, 0)

    return body(a_tab, b_tab, i_flat, j_flat)


def _tc_node(node_h, x, part, wts):
    n, node_dim = node_h.shape
    blk = 2000
    grid = n // blk

    def body(nh_ref, x_ref, part_ref, w1_ref, b1_ref, w2_ref, b2_ref,
             w3_ref, b3_ref, g_ref, beta_ref, wa_ref, wb_ref,
             out_h2, out_a, out_b):
        agg = (part_ref[0] + part_ref[1])[:, :16]
        h0 = nh_ref[...]
        w1 = w1_ref[...]
        z = _dot(h0, w1[:node_dim]) + _dot(agg, w1[node_dim:]) + b1_ref[...]
        z = _silu(z)
        z = _silu(_dot(z, w2_ref[...]) + b2_ref[...])
        h = _dot(z, w3_ref[...]) + b3_ref[...]
        h2 = _iln(h, g_ref[...], beta_ref[...]) + h0
        out_h2[...] = h2
        xb = x_ref[...]
        xpad = jnp.concatenate(
            [xb, jnp.zeros((blk, 61), jnp.float32)], axis=-1)
        out_a[...] = jnp.concatenate([_dot(h2, wa_ref[...]), -xpad], axis=-1)
        out_b[...] = jnp.concatenate([_dot(h2, wb_ref[...]), xpad], axis=-1)

    full = lambda s: pl.BlockSpec(s, lambda i: tuple(0 for _ in s))
    row = lambda s: pl.BlockSpec(s, lambda i: (i,) + tuple(0 for _ in s[1:]))
    w1, b1, w2, b2, w3, b3, g, beta, wa, wb = wts
    return pl.pallas_call(
        body,
        grid=(grid,),
        in_specs=[
            row((blk, node_dim)),
            row((blk, 3)),
            pl.BlockSpec((2, blk, 128), lambda i: (0, i, 0)),
            full(w1.shape), full(b1.shape), full(w2.shape), full(b2.shape),
            full(w3.shape), full(b3.shape), full(g.shape), full(beta.shape),
            full(wa.shape), full(wb.shape),
        ],
        out_specs=[
            row((blk, node_dim)), row((blk, 128)), row((blk, 128)),
        ],
        out_shape=[
            jax.ShapeDtypeStruct((n, node_dim), jnp.float32),
            jax.ShapeDtypeStruct((n, 128), jnp.float32),
            jax.ShapeDtypeStruct((n, 128), jnp.float32),
        ],
    )(node_h, x, part, w1, b1, w2, b2, w3, b3, g, beta, wa, wb)


def _tc_edge(sd, et, wts):
    e_total = sd.shape[0]
    blk = 3200
    grid = e_total // blk
    (wcat1, bias1, w2blk, bias2, w3blk, bias3,
     g, beta, xg, xbeta, eo_w, eo_bias) = wts

    def body(sd_ref, et_ref, wcat1_r, bias1_r, w2_r, bias2_r, w3_r, bias3_r,
             g_r, beta_r, xg_r, xbeta_r, eow_r, eobias_r, out_ref):
        sdb = sd_ref[...]
        s = sdb[:, :64]
        d16 = sdb[:, 64:80]
        w1c = wcat1_r[...][:16, :64]
        xw1 = wcat1_r[...][16:, 64:]
        u = _silu(s + _dot(et_ref[...], w1c) + bias1_r[...][:, :64])
        u = _silu(_dot(u, w2_r[...][:64, :64]) + bias2_r[...][:, :64])
        oh = _iln(_dot(u, w3_r[...][:64, :64]) + bias3_r[...][:, :64],
                  g_r[...], beta_r[...])
        dl = jnp.sqrt(jnp.sum(d16 * d16, axis=-1, keepdims=True) + 1e-12)
        lane = lax.broadcasted_iota(jnp.int32, d16.shape, 1)
        ex_in = jnp.where(lane == 3, dl, d16)
        v = _silu(_dot(ex_in, xw1) + bias1_r[...][:, 64:])
        v = _silu(_dot(v, w2_r[...][64:, 64:]) + bias2_r[...][:, 64:])
        ov = _iln(_dot(v, w3_r[...][64:, 64:]) + bias3_r[...][:, 64:],
                  xg_r[...], xbeta_r[...])
        out_ref[...] = (_dot(oh, eow_r[...][:64]) + _dot(ov, eow_r[...][64:])
                        + eobias_r[...])

    full = lambda s: pl.BlockSpec(s, lambda i: tuple(0 for _ in s))
    row = lambda s: pl.BlockSpec(s, lambda i: (i,) + tuple(0 for _ in s[1:]))
    return pl.pallas_call(
        body,
        grid=(grid,),
        in_specs=[row((blk, 128)), row((blk, 16))] + [full(w.shape) for w in (
            wcat1, bias1, w2blk, bias2, w3blk, bias3,
            g, beta, xg, xbeta, eo_w, eo_bias)],
        out_specs=[row((blk, 64))],
        out_shape=[jax.ShapeDtypeStruct((e_total, 64), jnp.float32)],
    )(sd, et, wcat1, bias1, w2blk, bias2, w3blk, bias3,
      g, beta, xg, xbeta, eo_w, eo_bias)[0]


def _tc_pos(x, part, wts):
    n = x.shape[0]
    blk = 2000
    grid = n // blk
    w1x, w1a, b1, w2, b2, w3, b3, g, beta = wts

    def body(x_ref, part_ref, w1x_r, w1a_r, b1_r, w2_r, b2_r, w3_r, b3_r,
             g_r, beta_r, out_ref):
        agg = (part_ref[0] + part_ref[1])[:, :64]
        xb = x_ref[...]
        z = _dot(xb, w1x_r[...]) + _dot(agg, w1a_r[...]) + b1_r[...]
        z = _silu(z)
        z = _silu(_dot(z, w2_r[...]) + b2_r[...])
        t = _dot(z, w3_r[...]) + b3_r[...]
        out_ref[...] = _iln(t, g_r[...], beta_r[...]) + xb

    full = lambda s: pl.BlockSpec(s, lambda i: tuple(0 for _ in s))
    row = lambda s: pl.BlockSpec(s, lambda i: (i,) + tuple(0 for _ in s[1:]))
    return pl.pallas_call(
        body,
        grid=(grid,),
        in_specs=[
            row((blk, 3)),
            pl.BlockSpec((2, blk, 128), lambda i: (0, i, 0)),
            full(w1x.shape), full(w1a.shape), full(b1.shape), full(w2.shape),
            full(b2.shape), full(w3.shape), full(b3.shape), full(g.shape),
            full(beta.shape),
        ],
        out_specs=[row((blk, 3))],
        out_shape=[jax.ShapeDtypeStruct((n, 3), jnp.float32)],
    )(x, part, w1x, w1a, b1, w2, b2, w3, b3, g, beta)[0]


def kernel(node_h, x, edge_index, edge_type_h, params):
    n, node_dim = node_h.shape
    e_total = edge_index.shape[1]
    i_flat = edge_index[0]
    j_flat = edge_index[1]

    r2 = lambda v: v.reshape(1, -1)

    part1 = _sc_segsum(edge_type_h, j_flat, n)

    ehw1 = params['eh_W'][0]
    node_wts = (
        params['n_W'][0], r2(params['n_b'][0]),
        params['n_W'][1], r2(params['n_b'][1]),
        params['n_W'][2], r2(params['n_b'][2]),
        r2(params['n_g']), r2(params['n_beta']),
        ehw1[:node_dim], ehw1[node_dim:2 * node_dim],
    )
    node_h2, a_tab, b_tab = _tc_node(node_h, x, part1, node_wts)

    sd = _sc_gather_add(a_tab, b_tab, i_flat, j_flat, 80)

    exw1 = params['ex_W'][0]
    xw1 = jnp.zeros((16, 64), jnp.float32).at[:4].set(exw1)
    def bdiag(a, b):
        za = jnp.zeros((a.shape[0], b.shape[1]), jnp.float32)
        zb = jnp.zeros((b.shape[0], a.shape[1]), jnp.float32)
        return jnp.concatenate([jnp.concatenate([a, za], -1),
                                jnp.concatenate([zb, b], -1)], 0)

    wcat1 = bdiag(ehw1[2 * node_dim:], xw1)
    w2blk = bdiag(params['eh_W'][1], params['ex_W'][1])
    w3blk = bdiag(params['eh_W'][2], params['ex_W'][2])
    cat2 = lambda a, b: jnp.concatenate([a, b]).reshape(1, -1)
    edge_wts = (
        wcat1, cat2(params['eh_b'][0], params['ex_b'][0]),
        w2blk, cat2(params['eh_b'][1], params['ex_b'][1]),
        w3blk, cat2(params['eh_b'][2], params['ex_b'][2]),
        r2(params['eh_g']), r2(params['eh_beta']),
        r2(params['ex_g']), r2(params['ex_beta']),
        params['eo_W'], r2(params['eo_b']),
    )
    edge_h = _tc_edge(sd, edge_type_h, edge_wts)

    part2 = _sc_segsum(edge_h, j_flat, n)

    pw1 = params['p_W'][0]
    pos_wts = (
        pw1[:3], pw1[3:], r2(params['p_b'][0]),
        params['p_W'][1], r2(params['p_b'][1]),
        params['p_W'][2], r2(params['p_b'][2]),
        r2(params['p_g']), r2(params['p_beta']),
    )
    x2 = _tc_pos(x, part2, pos_wts)

    return (edge_h, node_h2, x2)
